# Initial kernel scaffold; baseline (speedup 1.0000x reference)
import jax
import jax.numpy as jnp
from jax.experimental import pallas as pl


def kernel(h, key_buffer, previous_R, R0, R1):
    raise NotImplementedError("write your pallas kernel here")


# speedup vs baseline: 2.6833x; 2.6833x over previous
import functools

import jax
import jax.numpy as jnp
from jax import lax
from jax.experimental import pallas as pl
from jax.experimental.pallas import tpu as pltpu
from jax.experimental.pallas import tpu_sc as plsc

H = 128
U = 4
SUB = 32
BZ = 1024
UB = U * BZ
NKEY = 100000
KB = 1000


def _make_w(pr, r0, r1):
    return jnp.concatenate(
        [
            jnp.dot(pr[:, : H // 2], r0, preferred_element_type=jnp.float32),
            jnp.dot(pr[:, H // 2 :], r1, preferred_element_type=jnp.float32),
        ],
        axis=1,
    )


def _subspace_mask(shape):
    row_u = lax.broadcasted_iota(jnp.int32, shape, 0) // BZ
    col_u = lax.broadcasted_iota(jnp.int32, shape, 1) // SUB
    return row_u == col_u


def _search_body(h_ref, key_ref, pr_ref, r0_ref, r1_ref, qh_ref, idx_ref,
                 w_s, qbd_s, bv_s, bi_s):
    k = pl.program_id(0)
    nsteps = pl.num_programs(0)

    @pl.when(k == 0)
    def _init():
        w = _make_w(pr_ref[...], r0_ref[...], r1_ref[...])
        w_s[...] = w
        qh = jnp.dot(h_ref[...], w, preferred_element_type=jnp.float32)
        qh_ref[...] = qh
        qt = jnp.concatenate([qh, qh, qh, qh], axis=0)
        qbd = jnp.where(_subspace_mask((UB, H)), qt, 0.0)
        qn = jnp.sqrt(jnp.sum(qbd * qbd, axis=1, keepdims=True))
        qbd_s[...] = qbd / jnp.clip(qn, 1e-8, None)
        bv_s[...] = jnp.full((1, UB), -jnp.inf, jnp.float32)
        bi_s[...] = jnp.zeros((1, UB), jnp.int32)

    kp = jnp.dot(key_ref[...], w_s[...], preferred_element_type=jnp.float32)
    rg = lax.broadcasted_iota(jnp.int32, (H, H), 0) // SUB
    cg = lax.broadcasted_iota(jnp.int32, (H, H), 1) // SUB
    ones_bd = jnp.where(rg == cg, 1.0, 0.0)
    n2 = jnp.dot(kp * kp, ones_bd, preferred_element_type=jnp.float32)
    kn = kp / jnp.clip(jnp.sqrt(n2), 1e-3, None)
    sim = lax.dot_general(kn, qbd_s[...], (((1,), (1,)), ((), ())),
                          preferred_element_type=jnp.float32)
    bm = jnp.max(sim, axis=0, keepdims=True)
    rid = lax.broadcasted_iota(jnp.int32, sim.shape, 0)
    bi = jnp.min(jnp.where(sim == bm, rid, jnp.int32(NKEY)), axis=0,
                 keepdims=True) + k * KB
    upd = bm > bv_s[...]
    bv_s[...] = jnp.where(upd, bm, bv_s[...])
    bi_s[...] = jnp.where(upd, bi, bi_s[...])

    @pl.when(k == nsteps - 1)
    def _fin():
        idx_ref[...] = bi_s[...]


def _run_search(h, key_buffer, previous_R, R0, R1):
    nsteps = NKEY // KB
    return pl.pallas_call(
        _search_body,
        grid=(nsteps,),
        in_specs=[
            pl.BlockSpec((BZ, H), lambda k: (0, 0)),
            pl.BlockSpec((KB, H), lambda k: (k, 0)),
            pl.BlockSpec((H, H), lambda k: (0, 0)),
            pl.BlockSpec((H // 2, H // 2), lambda k: (0, 0)),
            pl.BlockSpec((H // 2, H // 2), lambda k: (0, 0)),
        ],
        out_specs=[
            pl.BlockSpec((BZ, H), lambda k: (0, 0)),
            pl.BlockSpec((1, UB), lambda k: (0, 0)),
        ],
        out_shape=[
            jax.ShapeDtypeStruct((BZ, H), jnp.float32),
            jax.ShapeDtypeStruct((1, UB), jnp.int32),
        ],
        scratch_shapes=[
            pltpu.VMEM((H, H), jnp.float32),
            pltpu.VMEM((UB, H), jnp.float32),
            pltpu.VMEM((1, UB), jnp.float32),
            pltpu.VMEM((1, UB), jnp.int32),
        ],
    )(h, key_buffer, previous_R, R0, R1)


def _run_gather(idx, key_buffer):
    info = plsc.get_sparse_core_info()
    nc, ns = info.num_cores, info.num_subcores
    nw = nc * ns
    bpw = UB // nw
    mesh = plsc.VectorSubcoreMesh(core_axis_name="c", subcore_axis_name="s")

    @functools.partial(
        pl.kernel,
        mesh=mesh,
        out_type=jax.ShapeDtypeStruct((UB, H), jnp.float32),
        scratch_types=[
            pltpu.VMEM((bpw,), jnp.int32),
            pltpu.VMEM((bpw, H), jnp.float32),
            pltpu.SemaphoreType.DMA,
        ],
    )
    def gather(idx_hbm, key_hbm, out_hbm, idx_v, rows_v, sem):
        wid = lax.axis_index("s") * nc + lax.axis_index("c")
        base = wid * bpw
        pltpu.sync_copy(idx_hbm.at[pl.ds(base, bpw)], idx_v)
        pltpu.async_copy(key_hbm.at[idx_v], rows_v, sem).wait()
        pltpu.sync_copy(rows_v, out_hbm.at[pl.ds(base, bpw)])

    return gather(idx, key_buffer)


def _loss_body(g_ref, qh_ref, pr_ref, r0_ref, r1_ref, out_ref):
    w = _make_w(pr_ref[...], r0_ref[...], r1_ref[...])
    gp = jnp.dot(g_ref[...], w, preferred_element_type=jnp.float32)
    qh = qh_ref[...]
    qt = jnp.concatenate([qh, qh, qh, qh], axis=0)
    mask = _subspace_mask((UB, H))
    qm = jnp.where(mask, qt, 0.0)
    gm = jnp.where(mask, gp, 0.0)
    num = jnp.sum(qm * gm, axis=1, keepdims=True)
    nq = jnp.sqrt(jnp.sum(qm * qm, axis=1, keepdims=True))
    ng = jnp.sqrt(jnp.sum(gm * gm, axis=1, keepdims=True))
    cos = num / (jnp.clip(nq, 1e-8, None) * jnp.clip(ng, 1e-8, None))
    out_ref[0, 0] = -jnp.sum(cos) / jnp.float32(UB)


def _run_loss(gathered, qh, previous_R, R0, R1):
    return pl.pallas_call(
        _loss_body,
        in_specs=[
            pl.BlockSpec((UB, H), lambda: (0, 0)),
            pl.BlockSpec((BZ, H), lambda: (0, 0)),
            pl.BlockSpec((H, H), lambda: (0, 0)),
            pl.BlockSpec((H // 2, H // 2), lambda: (0, 0)),
            pl.BlockSpec((H // 2, H // 2), lambda: (0, 0)),
        ],
        out_specs=pl.BlockSpec(memory_space=pltpu.SMEM),
        out_shape=jax.ShapeDtypeStruct((1, 1), jnp.float32),
    )(gathered, qh, previous_R, R0, R1)


def kernel(h, key_buffer, previous_R, R0, R1):
    qh, idx = _run_search(h, key_buffer, previous_R, R0, R1)
    gathered = _run_gather(idx.reshape(UB), key_buffer)
    loss = _run_loss(gathered, qh, previous_R, R0, R1)
    return loss[0, 0]


# speedup vs baseline: 2.8606x; 1.0661x over previous
import functools

import jax
import jax.numpy as jnp
from jax import lax
from jax.experimental import pallas as pl
from jax.experimental.pallas import tpu as pltpu
from jax.experimental.pallas import tpu_sc as plsc

H = 128
U = 4
SUB = 32
BZ = 1024
UB = U * BZ
NKEY = 100000
KB = 1000


def _make_w(pr, r0, r1):
    return jnp.concatenate(
        [
            jnp.dot(pr[:, : H // 2], r0, preferred_element_type=jnp.float32),
            jnp.dot(pr[:, H // 2 :], r1, preferred_element_type=jnp.float32),
        ],
        axis=1,
    )


def _subspace_mask(shape):
    row_u = lax.broadcasted_iota(jnp.int32, shape, 0) // BZ
    col_u = lax.broadcasted_iota(jnp.int32, shape, 1) // SUB
    return row_u == col_u


def _search_body(h_ref, key_ref, pr_ref, r0_ref, r1_ref, qh_ref, idx_ref,
                 w_s, qbd_s, bv_s, bi_s):
    k = pl.program_id(0)
    nsteps = pl.num_programs(0)

    @pl.when(k == 0)
    def _init():
        w = _make_w(pr_ref[...], r0_ref[...], r1_ref[...])
        w_s[...] = w
        qh = jnp.dot(h_ref[...], w, preferred_element_type=jnp.float32)
        qh_ref[...] = qh
        qt = jnp.concatenate([qh, qh, qh, qh], axis=0)
        qbd = jnp.where(_subspace_mask((UB, H)), qt, 0.0)
        qn = jnp.sqrt(jnp.sum(qbd * qbd, axis=1, keepdims=True))
        qbd_s[...] = (qbd / jnp.clip(qn, 1e-8, None)).astype(jnp.bfloat16)
        bv_s[...] = jnp.full((1, UB), -jnp.inf, jnp.float32)
        bi_s[...] = jnp.zeros((1, UB), jnp.int32)

    kp = jnp.dot(key_ref[...], w_s[...], preferred_element_type=jnp.float32)
    rg = lax.broadcasted_iota(jnp.int32, (H, H), 0) // SUB
    cg = lax.broadcasted_iota(jnp.int32, (H, H), 1) // SUB
    ones_bd = jnp.where(rg == cg, 1.0, 0.0)
    n2 = jnp.dot(kp * kp, ones_bd, preferred_element_type=jnp.float32)
    kn = (kp / jnp.clip(jnp.sqrt(n2), 1e-3, None)).astype(jnp.bfloat16)
    sim = lax.dot_general(kn, qbd_s[...], (((1,), (1,)), ((), ())),
                          preferred_element_type=jnp.float32)
    bm = jnp.max(sim, axis=0, keepdims=True)
    rid = lax.broadcasted_iota(jnp.int32, sim.shape, 0)
    bi = jnp.min(jnp.where(sim == bm, rid, jnp.int32(NKEY)), axis=0,
                 keepdims=True) + k * KB
    upd = bm > bv_s[...]
    bv_s[...] = jnp.where(upd, bm, bv_s[...])
    bi_s[...] = jnp.where(upd, bi, bi_s[...])

    @pl.when(k == nsteps - 1)
    def _fin():
        idx_ref[...] = bi_s[...]


def _run_search(h, key_buffer, previous_R, R0, R1):
    nsteps = NKEY // KB
    return pl.pallas_call(
        _search_body,
        grid=(nsteps,),
        in_specs=[
            pl.BlockSpec((BZ, H), lambda k: (0, 0)),
            pl.BlockSpec((KB, H), lambda k: (k, 0)),
            pl.BlockSpec((H, H), lambda k: (0, 0)),
            pl.BlockSpec((H // 2, H // 2), lambda k: (0, 0)),
            pl.BlockSpec((H // 2, H // 2), lambda k: (0, 0)),
        ],
        out_specs=[
            pl.BlockSpec((BZ, H), lambda k: (0, 0)),
            pl.BlockSpec((1, UB), lambda k: (0, 0)),
        ],
        out_shape=[
            jax.ShapeDtypeStruct((BZ, H), jnp.float32),
            jax.ShapeDtypeStruct((1, UB), jnp.int32),
        ],
        scratch_shapes=[
            pltpu.VMEM((H, H), jnp.float32),
            pltpu.VMEM((UB, H), jnp.bfloat16),
            pltpu.VMEM((1, UB), jnp.float32),
            pltpu.VMEM((1, UB), jnp.int32),
        ],
    )(h, key_buffer, previous_R, R0, R1)


def _run_gather(idx, key_buffer):
    info = plsc.get_sparse_core_info()
    nc, ns = info.num_cores, info.num_subcores
    nw = nc * ns
    bpw = UB // nw
    mesh = plsc.VectorSubcoreMesh(core_axis_name="c", subcore_axis_name="s")

    @functools.partial(
        pl.kernel,
        mesh=mesh,
        out_type=jax.ShapeDtypeStruct((UB, H), jnp.float32),
        scratch_types=[
            pltpu.VMEM((bpw,), jnp.int32),
            pltpu.VMEM((bpw, H), jnp.float32),
            pltpu.SemaphoreType.DMA,
        ],
    )
    def gather(idx_hbm, key_hbm, out_hbm, idx_v, rows_v, sem):
        wid = lax.axis_index("s") * nc + lax.axis_index("c")
        base = wid * bpw
        pltpu.sync_copy(idx_hbm.at[pl.ds(base, bpw)], idx_v)
        pltpu.async_copy(key_hbm.at[idx_v], rows_v, sem).wait()
        pltpu.sync_copy(rows_v, out_hbm.at[pl.ds(base, bpw)])

    return gather(idx, key_buffer)


def _loss_body(g_ref, qh_ref, pr_ref, r0_ref, r1_ref, out_ref):
    w = _make_w(pr_ref[...], r0_ref[...], r1_ref[...])
    gp = jnp.dot(g_ref[...], w, preferred_element_type=jnp.float32)
    qh = qh_ref[...]
    qt = jnp.concatenate([qh, qh, qh, qh], axis=0)
    mask = _subspace_mask((UB, H))
    qm = jnp.where(mask, qt, 0.0)
    gm = jnp.where(mask, gp, 0.0)
    num = jnp.sum(qm * gm, axis=1, keepdims=True)
    nq = jnp.sqrt(jnp.sum(qm * qm, axis=1, keepdims=True))
    ng = jnp.sqrt(jnp.sum(gm * gm, axis=1, keepdims=True))
    cos = num / (jnp.clip(nq, 1e-8, None) * jnp.clip(ng, 1e-8, None))
    out_ref[0, 0] = -jnp.sum(cos) / jnp.float32(UB)


def _run_loss(gathered, qh, previous_R, R0, R1):
    return pl.pallas_call(
        _loss_body,
        in_specs=[
            pl.BlockSpec((UB, H), lambda: (0, 0)),
            pl.BlockSpec((BZ, H), lambda: (0, 0)),
            pl.BlockSpec((H, H), lambda: (0, 0)),
            pl.BlockSpec((H // 2, H // 2), lambda: (0, 0)),
            pl.BlockSpec((H // 2, H // 2), lambda: (0, 0)),
        ],
        out_specs=pl.BlockSpec(memory_space=pltpu.SMEM),
        out_shape=jax.ShapeDtypeStruct((1, 1), jnp.float32),
    )(gathered, qh, previous_R, R0, R1)


def kernel(h, key_buffer, previous_R, R0, R1):
    qh, idx = _run_search(h, key_buffer, previous_R, R0, R1)
    gathered = _run_gather(idx.reshape(UB), key_buffer)
    loss = _run_loss(gathered, qh, previous_R, R0, R1)
    return loss[0, 0]


# speedup vs baseline: 3.6759x; 1.2850x over previous
import functools

import jax
import jax.numpy as jnp
from jax import lax
from jax.experimental import pallas as pl
from jax.experimental.pallas import tpu as pltpu
from jax.experimental.pallas import tpu_sc as plsc

H = 128
U = 4
SUB = 32
BZ = 1024
UB = U * BZ
NKEY = 100000
KB = 1000
IDXBITS = 10
IDXMASK = (1 << IDXBITS) - 1


def _make_w(pr, r0, r1):
    return jnp.concatenate(
        [
            jnp.dot(pr[:, : H // 2], r0, preferred_element_type=jnp.float32),
            jnp.dot(pr[:, H // 2 :], r1, preferred_element_type=jnp.float32),
        ],
        axis=1,
    )


def _subspace_mask(shape):
    row_u = lax.broadcasted_iota(jnp.int32, shape, 0) // BZ
    col_u = lax.broadcasted_iota(jnp.int32, shape, 1) // SUB
    return row_u == col_u


def _prep_body(h_ref, pr_ref, r0_ref, r1_ref, w_ref, qh_ref, qbd_ref, e_ref):
    w = _make_w(pr_ref[...], r0_ref[...], r1_ref[...])
    w_ref[...] = w
    qh = jnp.dot(h_ref[...], w, preferred_element_type=jnp.float32)
    qh_ref[...] = qh
    qt = jnp.concatenate([qh, qh, qh, qh], axis=0)
    qbd = jnp.where(_subspace_mask((UB, H)), qt, 0.0)
    qn = jnp.sqrt(jnp.sum(qbd * qbd, axis=1, keepdims=True))
    qbd_ref[...] = (qbd / jnp.clip(qn, 1e-8, None)).astype(jnp.bfloat16)
    rg = lax.broadcasted_iota(jnp.int32, (H, H), 0) // SUB
    cg = lax.broadcasted_iota(jnp.int32, (H, H), 1) // SUB
    e_ref[...] = jnp.where(rg == cg, 1.0, 0.0)


def _run_prep(h, previous_R, R0, R1):
    return pl.pallas_call(
        _prep_body,
        out_shape=[
            jax.ShapeDtypeStruct((H, H), jnp.float32),
            jax.ShapeDtypeStruct((BZ, H), jnp.float32),
            jax.ShapeDtypeStruct((UB, H), jnp.bfloat16),
            jax.ShapeDtypeStruct((H, H), jnp.float32),
        ],
    )(h, previous_R, R0, R1)


def _search_body(key_ref, w_ref, qbd_ref, e_ref, idx_ref, bp_s, bb_s):
    k = pl.program_id(0)
    nsteps = pl.num_programs(0)

    @pl.when(k == 0)
    def _init():
        bp_s[...] = jnp.zeros((1, UB), jnp.int32)
        bb_s[...] = jnp.zeros((1, UB), jnp.int32)

    kp = jnp.dot(key_ref[...], w_ref[...], preferred_element_type=jnp.float32)
    n2 = jnp.dot(kp * kp, e_ref[...], preferred_element_type=jnp.float32)
    inv = jnp.minimum(lax.rsqrt(n2), 1e3)
    kn = (kp * inv).astype(jnp.bfloat16)
    sim = lax.dot_general(kn, qbd_ref[...], (((1,), (1,)), ((), ())),
                          preferred_element_type=jnp.float32)
    t = lax.bitcast_convert_type(sim + 2.0, jnp.int32)
    rowc = IDXMASK - lax.broadcasted_iota(jnp.int32, (KB, 1), 0)
    packed = (t & jnp.int32(~IDXMASK)) | rowc
    p = jnp.max(packed, axis=0, keepdims=True)
    upd = (p & jnp.int32(~IDXMASK)) > (bp_s[...] & jnp.int32(~IDXMASK))
    bp_s[...] = jnp.where(upd, p, bp_s[...])
    bb_s[...] = jnp.where(upd, k, bb_s[...])

    @pl.when(k == nsteps - 1)
    def _fin():
        local = IDXMASK - (bp_s[...] & jnp.int32(IDXMASK))
        idx_ref[...] = bb_s[...] * KB + local


def _run_search(key_buffer, w, qbd, e):
    nsteps = NKEY // KB
    return pl.pallas_call(
        _search_body,
        grid=(nsteps,),
        in_specs=[
            pl.BlockSpec((KB, H), lambda k: (k, 0)),
            pl.BlockSpec((H, H), lambda k: (0, 0)),
            pl.BlockSpec((UB, H), lambda k: (0, 0)),
            pl.BlockSpec((H, H), lambda k: (0, 0)),
        ],
        out_specs=pl.BlockSpec((1, UB), lambda k: (0, 0)),
        out_shape=jax.ShapeDtypeStruct((1, UB), jnp.int32),
        scratch_shapes=[
            pltpu.VMEM((1, UB), jnp.int32),
            pltpu.VMEM((1, UB), jnp.int32),
        ],
    )(key_buffer, w, qbd, e)


def _run_gather(idx, key_buffer):
    info = plsc.get_sparse_core_info()
    nc, ns = info.num_cores, info.num_subcores
    nw = nc * ns
    bpw = UB // nw
    mesh = plsc.VectorSubcoreMesh(core_axis_name="c", subcore_axis_name="s")

    @functools.partial(
        pl.kernel,
        mesh=mesh,
        out_type=jax.ShapeDtypeStruct((UB, H), jnp.float32),
        scratch_types=[
            pltpu.VMEM((bpw,), jnp.int32),
            pltpu.VMEM((bpw, H), jnp.float32),
            pltpu.SemaphoreType.DMA,
        ],
    )
    def gather(idx_hbm, key_hbm, out_hbm, idx_v, rows_v, sem):
        wid = lax.axis_index("s") * nc + lax.axis_index("c")
        base = wid * bpw
        pltpu.sync_copy(idx_hbm.at[pl.ds(base, bpw)], idx_v)
        pltpu.async_copy(key_hbm.at[idx_v], rows_v, sem).wait()
        pltpu.sync_copy(rows_v, out_hbm.at[pl.ds(base, bpw)])

    return gather(idx, key_buffer)


def _loss_body(g_ref, qh_ref, w_ref, out_ref):
    gp = jnp.dot(g_ref[...], w_ref[...], preferred_element_type=jnp.float32)
    qh = qh_ref[...]
    qt = jnp.concatenate([qh, qh, qh, qh], axis=0)
    mask = _subspace_mask((UB, H))
    qm = jnp.where(mask, qt, 0.0)
    gm = jnp.where(mask, gp, 0.0)
    num = jnp.sum(qm * gm, axis=1, keepdims=True)
    nq = jnp.sqrt(jnp.sum(qm * qm, axis=1, keepdims=True))
    ng = jnp.sqrt(jnp.sum(gm * gm, axis=1, keepdims=True))
    cos = num / (jnp.clip(nq, 1e-8, None) * jnp.clip(ng, 1e-8, None))
    out_ref[0, 0] = -jnp.sum(cos) / jnp.float32(UB)


def _run_loss(gathered, qh, w):
    return pl.pallas_call(
        _loss_body,
        out_specs=pl.BlockSpec(memory_space=pltpu.SMEM),
        out_shape=jax.ShapeDtypeStruct((1, 1), jnp.float32),
    )(gathered, qh, w)


def kernel(h, key_buffer, previous_R, R0, R1):
    w, qh, qbd, e = _run_prep(h, previous_R, R0, R1)
    idx = _run_search(key_buffer, w, qbd, e)
    gathered = _run_gather(idx.reshape(UB), key_buffer)
    loss = _run_loss(gathered, qh, w)
    return loss[0, 0]
